# baseline (speedup 1.0000x reference)
import functools

import jax
import jax.numpy as jnp
from jax import lax
from jax.experimental import pallas as pl
from jax.experimental.pallas import tpu as pltpu
from jax.experimental.pallas import tpu_sc as plsc

N = 10000
D = 128
H = 64
E = 320000
NPAD = 10240
NC = 2
NS = 16
NW = NC * NS
EPW = E // NW
BE = 112
EPWP = 10080
NBLK = EPWP // BE
RB = 1024
NRB = NPAD // RB

VIEW_RELS = [3, 2, 2, 2, 1, 1, 1]
NREL = 12
VIEW_OF_REL = [0, 0, 0, 1, 1, 2, 2, 3, 3, 4, 5, 6]
REL_START = [0, 3, 5, 7, 9, 10, 11]


def _mm_body(x_ref, w_ref, y_ref):
    y_ref[...] = jnp.dot(x_ref[0], w_ref[0],
                         preferred_element_type=jnp.float32)[None]


def _rel_matmul(x_pad, w_rel):
    return pl.pallas_call(
        _mm_body,
        grid=(NREL, NRB),
        in_specs=[
            pl.BlockSpec(
                (1, RB, D),
                lambda r, b: (jnp.where(r < 3, 0,
                                        jnp.where(r < 9, (r - 3) // 2 + 1,
                                                  r - 5)), b, 0)),
            pl.BlockSpec((1, D, H), lambda r, b: (r, 0, 0)),
        ],
        out_specs=pl.BlockSpec((1, RB, H), lambda r, b: (r, b, 0)),
        out_shape=jax.ShapeDtypeStruct((NREL, NPAD, H), jnp.float32),
    )(x_pad, w_rel)


def _sc_body(src_hbm, dst_hbm, y_hbm, z2d_hbm, z1d_hbm, ones_hbm,
             agg_out, deg_out,
             acc_sh, deg_sh, srcb, dstb, rows, ones, sem, ssem, dsem):
    c = lax.axis_index("c")
    s = lax.axis_index("s")
    wid = c * NS + s
    rows_per_tile = NPAD // NS
    my = pl.ds(s * rows_per_tile, rows_per_tile)

    pltpu.sync_copy(ones_hbm, ones)
    pltpu.sync_copy(z2d_hbm, acc_sh.at[my])
    pltpu.sync_copy(z1d_hbm, deg_sh.at[my])
    for (zv, zk) in ((1, 2), (2, 2), (3, 2), (4, 1), (4, 2),
                     (5, 1), (5, 2), (6, 1), (6, 2)):
        pltpu.sync_copy(acc_sh.at[my], agg_out.at[zv, zk, c, my])
        pltpu.sync_copy(deg_sh.at[my], deg_out.at[zv, zk, c, my])
    plsc.subcore_barrier()

    def rel_body(r, carry):
        pltpu.sync_copy(src_hbm.at[r, wid], srcb)
        pltpu.sync_copy(dst_hbm.at[r, wid], dstb)

        pltpu.async_copy(y_hbm.at[srcb.at[0]], rows.at[0], sem)
        pltpu.async_copy(y_hbm.at[srcb.at[1]], rows.at[1], sem)

        def blk_body(j, carry2):
            b = j % 3
            pltpu.make_async_copy(y_hbm.at[srcb.at[j]], rows.at[b], sem).wait()
            pltpu.async_copy(rows.at[b], acc_sh.at[dstb.at[j]], ssem,
                             add=True)
            pltpu.async_copy(ones.at[j], deg_sh.at[dstb.at[j]], ssem,
                             add=True)

            @pl.when(j + 2 < NBLK)
            def _():
                @pl.when(j >= 1)
                def _():
                    pltpu.make_async_copy(
                        rows.at[(j - 1) % 3],
                        acc_sh.at[dstb.at[j - 1]], ssem).wait()
                    pltpu.make_async_copy(
                        ones.at[j - 1], deg_sh.at[dstb.at[j - 1]],
                        ssem).wait()
                pltpu.async_copy(y_hbm.at[srcb.at[j + 2]],
                                 rows.at[(j + 2) % 3], sem)

            return carry2

        lax.fori_loop(0, NBLK, blk_body, 0)
        for t in (NBLK - 3, NBLK - 2, NBLK - 1):
            pltpu.make_async_copy(rows.at[t % 3],
                                  acc_sh.at[dstb.at[t]], ssem).wait()
            pltpu.make_async_copy(ones.at[t], deg_sh.at[dstb.at[t]],
                                  ssem).wait()
        plsc.subcore_barrier()

        v_r = jnp.where(r < 3, 0,
                        jnp.where(r < 9, (r - 3) // 2 + 1, r - 5))
        k_r = jnp.where(r < 3, r, jnp.where(r < 9, (r - 3) % 2, 0))
        pltpu.sync_copy(acc_sh.at[my], agg_out.at[v_r, k_r, c, my])
        pltpu.sync_copy(deg_sh.at[my], deg_out.at[v_r, k_r, c, my])
        pltpu.sync_copy(z2d_hbm, acc_sh.at[my])
        pltpu.sync_copy(z1d_hbm, deg_sh.at[my])
        plsc.subcore_barrier()
        return carry

    lax.fori_loop(0, NREL, rel_body, 0)


def _sc_segment_sums(src_all, dst_all, y_flat):
    rows_per_tile = NPAD // NS
    mesh = plsc.VectorSubcoreMesh(core_axis_name="c", subcore_axis_name="s",
                                  num_cores=NC, num_subcores=NS)
    kern = pl.kernel(
        _sc_body,
        out_type=(
            jax.ShapeDtypeStruct((7, 3, NC, NPAD, H), jnp.float32),
            jax.ShapeDtypeStruct((7, 3, NC, NPAD), jnp.float32),
        ),
        mesh=mesh,
        scratch_types=[
            pltpu.VMEM_SHARED((NPAD, H), jnp.float32),
            pltpu.VMEM_SHARED((NPAD,), jnp.float32),
            pltpu.VMEM((NBLK, BE), jnp.int32),
            pltpu.VMEM((NBLK, BE), jnp.int32),
            pltpu.VMEM((3, BE, H), jnp.float32),
            pltpu.VMEM((NBLK, BE), jnp.float32),
            pltpu.SemaphoreType.DMA,
            pltpu.SemaphoreType.DMA,
            pltpu.SemaphoreType.DMA,
        ],
        compiler_params=pltpu.CompilerParams(use_tc_tiling_on_sc=False),
    )
    z2d = jnp.zeros((rows_per_tile, H), jnp.float32)
    z1d = jnp.zeros((rows_per_tile,), jnp.float32)
    ones = jnp.ones((NBLK, BE), jnp.float32)
    return kern(src_all, dst_all, y_flat, z2d, z1d, ones)


def _fin_body(x_ref, ws_ref, agg_ref, deg_ref, out_ref):
    h = jnp.dot(x_ref[0], ws_ref[0], preferred_element_type=jnp.float32)
    for k in range(3):
        a = agg_ref[0, k, 0] + agg_ref[0, k, 1]
        d = deg_ref[0, k, 0] + deg_ref[0, k, 1]
        h = h + a / jnp.maximum(d, 1.0)[:, None]
    out_ref[0] = h * (1.0 / (1.0 + jnp.exp(-h)))


def _finish_all(x_pad, w_self_all, agg, deg):
    return pl.pallas_call(
        _fin_body,
        grid=(7, NRB),
        in_specs=[
            pl.BlockSpec((1, RB, D), lambda v, b: (v, b, 0)),
            pl.BlockSpec((1, D, H), lambda v, b: (v, 0, 0)),
            pl.BlockSpec((1, 3, NC, RB, H), lambda v, b: (v, 0, 0, b, 0)),
            pl.BlockSpec((1, 3, NC, RB), lambda v, b: (v, 0, 0, b)),
        ],
        out_specs=pl.BlockSpec((1, RB, H), lambda v, b: (v, b, 0)),
        out_shape=jax.ShapeDtypeStruct((7, NPAD, H), jnp.float32),
    )(x_pad, w_self_all, agg, deg)


def kernel(x_coarse, ei_coarse_gene, ei_coarse_methy, ei_coarse_mirna,
           W_coarse_self, W_coarse_gene, W_coarse_methy, W_coarse_mirna,
           x_medium1, ei_medium1_gene, ei_medium1_methy,
           W_medium1_self, W_medium1_gene, W_medium1_methy,
           x_medium2, ei_medium2_gene, ei_medium2_mirna,
           W_medium2_self, W_medium2_gene, W_medium2_mirna,
           x_medium3, ei_medium3_methy, ei_medium3_mirna,
           W_medium3_self, W_medium3_methy, W_medium3_mirna,
           x_fine1, ei_fine1_gene, W_fine1_self, W_fine1_gene,
           x_fine2, ei_fine2_methy, W_fine2_self, W_fine2_methy,
           x_fine3, ei_fine3_mirna, W_fine3_self, W_fine3_mirna):
    xs = [x_coarse, x_medium1, x_medium2, x_medium3, x_fine1, x_fine2, x_fine3]
    eis = [ei_coarse_gene, ei_coarse_methy, ei_coarse_mirna,
           ei_medium1_gene, ei_medium1_methy,
           ei_medium2_gene, ei_medium2_mirna,
           ei_medium3_methy, ei_medium3_mirna,
           ei_fine1_gene, ei_fine2_methy, ei_fine3_mirna]
    w_selfs = [W_coarse_self, W_medium1_self, W_medium2_self, W_medium3_self,
               W_fine1_self, W_fine2_self, W_fine3_self]
    w_rels = [W_coarse_gene, W_coarse_methy, W_coarse_mirna,
              W_medium1_gene, W_medium1_methy,
              W_medium2_gene, W_medium2_mirna,
              W_medium3_methy, W_medium3_mirna,
              W_fine1_gene, W_fine2_methy, W_fine3_mirna]

    x_pad = jnp.pad(jnp.stack(xs), ((0, 0), (0, NPAD - N), (0, 0)))
    w_rel = jnp.stack(w_rels)

    y = _rel_matmul(x_pad, w_rel)
    y_flat = y.reshape(NREL * NPAD, H)

    ei = jnp.stack(eis)
    rel_off = (jnp.arange(NREL, dtype=jnp.int32) * NPAD)[:, None, None]
    src = ei[:, 0, :].reshape(NREL, NW, EPW)
    dst = ei[:, 1, :].reshape(NREL, NW, EPW)
    pad = ((0, 0), (0, 0), (0, EPWP - EPW))
    src_all = (jnp.pad(src, pad) + rel_off).reshape(NREL, NW, NBLK, BE)
    dst_pad = N + jnp.arange(EPWP - EPW, dtype=jnp.int32)
    dst_p = jnp.concatenate(
        [dst, jnp.broadcast_to(dst_pad, (NREL, NW, EPWP - EPW))], axis=2)
    dst_all = dst_p.reshape(NREL, NW, NBLK, BE)

    agg, deg = _sc_segment_sums(src_all, dst_all, y_flat)

    out = _finish_all(x_pad, jnp.stack(w_selfs), agg, deg)
    return out[:, :N, :]


# speedup vs baseline: 1.3308x; 1.3308x over previous
import functools

import jax
import jax.numpy as jnp
from jax import lax
from jax.experimental import pallas as pl
from jax.experimental.pallas import tpu as pltpu
from jax.experimental.pallas import tpu_sc as plsc

N = 10000
D = 128
H = 64
E = 320000
NPAD = 10240
NC = 2
NS = 16
NW = NC * NS
EPW = E // NW
BE = 80
EPWP = 10000
NBLK = EPWP // BE
RB = 1024
NRB = NPAD // RB

VIEW_RELS = [3, 2, 2, 2, 1, 1, 1]
NREL = 12
VIEW_OF_REL = [0, 0, 0, 1, 1, 2, 2, 3, 3, 4, 5, 6]
REL_START = [0, 3, 5, 7, 9, 10, 11]


def _mm_body(x_ref, w_ref, y_ref):
    y_ref[...] = jnp.dot(x_ref[0], w_ref[0],
                         preferred_element_type=jnp.float32)[None]


def _rel_matmul(x_pad, w_rel):
    return pl.pallas_call(
        _mm_body,
        grid=(NREL, NRB),
        in_specs=[
            pl.BlockSpec(
                (1, RB, D),
                lambda r, b: (jnp.where(r < 3, 0,
                                        jnp.where(r < 9, (r - 3) // 2 + 1,
                                                  r - 5)), b, 0)),
            pl.BlockSpec((1, D, H), lambda r, b: (r, 0, 0)),
        ],
        out_specs=pl.BlockSpec((1, RB, H), lambda r, b: (r, b, 0)),
        out_shape=jax.ShapeDtypeStruct((NREL, NPAD, H), jnp.float32),
    )(x_pad, w_rel)


def _sc_body(src_hbm, dst_hbm, y_hbm, z2d_hbm, z1d_hbm, ones_hbm,
             agg_out, deg_out,
             acc_sh, deg_sh, srcb, dstb, rows, ones, sem, ssem, dsem):
    c = lax.axis_index("c")
    s = lax.axis_index("s")
    wid = c * NS + s
    rows_per_tile = NPAD // NS
    my = pl.ds(s * rows_per_tile, rows_per_tile)

    pltpu.sync_copy(ones_hbm, ones)
    pltpu.sync_copy(z2d_hbm, acc_sh.at[my])
    pltpu.sync_copy(z1d_hbm, deg_sh.at[my])
    for (zv, zk) in ((1, 2), (2, 2), (3, 2), (4, 1), (4, 2),
                     (5, 1), (5, 2), (6, 1), (6, 2)):
        pltpu.sync_copy(acc_sh.at[my], agg_out.at[zv, zk, c, my])
        pltpu.sync_copy(deg_sh.at[my], deg_out.at[zv, zk, c, my])
    plsc.subcore_barrier()

    def rel_body(r, carry):
        pltpu.sync_copy(src_hbm.at[r, wid], srcb)
        pltpu.sync_copy(dst_hbm.at[r, wid], dstb)

        pltpu.async_copy(y_hbm.at[srcb.at[0]], rows.at[0], sem)
        pltpu.async_copy(y_hbm.at[srcb.at[1]], rows.at[1], sem)

        def blk_body(j, carry2):
            b = j % 3
            pltpu.make_async_copy(y_hbm.at[srcb.at[j]], rows.at[b], sem).wait()
            pltpu.async_copy(rows.at[b], acc_sh.at[dstb.at[j]], ssem,
                             add=True)
            pltpu.async_copy(ones.at[j], deg_sh.at[dstb.at[j]], ssem,
                             add=True)

            @pl.when(j + 2 < NBLK)
            def _():
                @pl.when(j >= 1)
                def _():
                    pltpu.make_async_copy(
                        rows.at[(j - 1) % 3],
                        acc_sh.at[dstb.at[j - 1]], ssem).wait()
                    pltpu.make_async_copy(
                        ones.at[j - 1], deg_sh.at[dstb.at[j - 1]],
                        ssem).wait()
                pltpu.async_copy(y_hbm.at[srcb.at[j + 2]],
                                 rows.at[(j + 2) % 3], sem)

            return carry2

        lax.fori_loop(0, NBLK, blk_body, 0)
        for t in (NBLK - 3, NBLK - 2, NBLK - 1):
            pltpu.make_async_copy(rows.at[t % 3],
                                  acc_sh.at[dstb.at[t]], ssem).wait()
            pltpu.make_async_copy(ones.at[t], deg_sh.at[dstb.at[t]],
                                  ssem).wait()
        plsc.subcore_barrier()

        v_r = jnp.where(r < 3, 0,
                        jnp.where(r < 9, (r - 3) // 2 + 1, r - 5))
        k_r = jnp.where(r < 3, r, jnp.where(r < 9, (r - 3) % 2, 0))
        pltpu.sync_copy(acc_sh.at[my], agg_out.at[v_r, k_r, c, my])
        pltpu.sync_copy(deg_sh.at[my], deg_out.at[v_r, k_r, c, my])
        pltpu.sync_copy(z2d_hbm, acc_sh.at[my])
        pltpu.sync_copy(z1d_hbm, deg_sh.at[my])
        plsc.subcore_barrier()
        return carry

    lax.fori_loop(0, NREL, rel_body, 0)


def _sc_segment_sums(src_all, dst_all, y_flat):
    rows_per_tile = NPAD // NS
    mesh = plsc.VectorSubcoreMesh(core_axis_name="c", subcore_axis_name="s",
                                  num_cores=NC, num_subcores=NS)
    kern = pl.kernel(
        _sc_body,
        out_type=(
            jax.ShapeDtypeStruct((7, 3, NC, NPAD, H), jnp.float32),
            jax.ShapeDtypeStruct((7, 3, NC, NPAD), jnp.float32),
        ),
        mesh=mesh,
        scratch_types=[
            pltpu.VMEM_SHARED((NPAD, H), jnp.float32),
            pltpu.VMEM_SHARED((NPAD,), jnp.float32),
            pltpu.VMEM((NBLK, BE), jnp.int32),
            pltpu.VMEM((NBLK, BE), jnp.int32),
            pltpu.VMEM((3, BE, H), jnp.float32),
            pltpu.VMEM((NBLK, BE), jnp.float32),
            pltpu.SemaphoreType.DMA,
            pltpu.SemaphoreType.DMA,
            pltpu.SemaphoreType.DMA,
        ],
        compiler_params=pltpu.CompilerParams(use_tc_tiling_on_sc=False),
    )
    z2d = jnp.zeros((rows_per_tile, H), jnp.float32)
    z1d = jnp.zeros((rows_per_tile,), jnp.float32)
    ones = jnp.ones((NBLK, BE), jnp.float32)
    return kern(src_all, dst_all, y_flat, z2d, z1d, ones)


def _fin_body(x_ref, ws_ref, agg_ref, deg_ref, out_ref):
    h = jnp.dot(x_ref[0], ws_ref[0], preferred_element_type=jnp.float32)
    for k in range(3):
        a = agg_ref[0, k, 0] + agg_ref[0, k, 1]
        d = deg_ref[0, k, 0] + deg_ref[0, k, 1]
        h = h + a / jnp.maximum(d, 1.0)[:, None]
    out_ref[0] = h * (1.0 / (1.0 + jnp.exp(-h)))


def _finish_all(x_pad, w_self_all, agg, deg):
    return pl.pallas_call(
        _fin_body,
        grid=(7, NRB),
        in_specs=[
            pl.BlockSpec((1, RB, D), lambda v, b: (v, b, 0)),
            pl.BlockSpec((1, D, H), lambda v, b: (v, 0, 0)),
            pl.BlockSpec((1, 3, NC, RB, H), lambda v, b: (v, 0, 0, b, 0)),
            pl.BlockSpec((1, 3, NC, RB), lambda v, b: (v, 0, 0, b)),
        ],
        out_specs=pl.BlockSpec((1, RB, H), lambda v, b: (v, b, 0)),
        out_shape=jax.ShapeDtypeStruct((7, NPAD, H), jnp.float32),
    )(x_pad, w_self_all, agg, deg)


def kernel(x_coarse, ei_coarse_gene, ei_coarse_methy, ei_coarse_mirna,
           W_coarse_self, W_coarse_gene, W_coarse_methy, W_coarse_mirna,
           x_medium1, ei_medium1_gene, ei_medium1_methy,
           W_medium1_self, W_medium1_gene, W_medium1_methy,
           x_medium2, ei_medium2_gene, ei_medium2_mirna,
           W_medium2_self, W_medium2_gene, W_medium2_mirna,
           x_medium3, ei_medium3_methy, ei_medium3_mirna,
           W_medium3_self, W_medium3_methy, W_medium3_mirna,
           x_fine1, ei_fine1_gene, W_fine1_self, W_fine1_gene,
           x_fine2, ei_fine2_methy, W_fine2_self, W_fine2_methy,
           x_fine3, ei_fine3_mirna, W_fine3_self, W_fine3_mirna):
    xs = [x_coarse, x_medium1, x_medium2, x_medium3, x_fine1, x_fine2, x_fine3]
    eis = [ei_coarse_gene, ei_coarse_methy, ei_coarse_mirna,
           ei_medium1_gene, ei_medium1_methy,
           ei_medium2_gene, ei_medium2_mirna,
           ei_medium3_methy, ei_medium3_mirna,
           ei_fine1_gene, ei_fine2_methy, ei_fine3_mirna]
    w_selfs = [W_coarse_self, W_medium1_self, W_medium2_self, W_medium3_self,
               W_fine1_self, W_fine2_self, W_fine3_self]
    w_rels = [W_coarse_gene, W_coarse_methy, W_coarse_mirna,
              W_medium1_gene, W_medium1_methy,
              W_medium2_gene, W_medium2_mirna,
              W_medium3_methy, W_medium3_mirna,
              W_fine1_gene, W_fine2_methy, W_fine3_mirna]

    x_pad = jnp.pad(jnp.stack(xs), ((0, 0), (0, NPAD - N), (0, 0)))
    w_rel = jnp.stack(w_rels)

    y = _rel_matmul(x_pad, w_rel)
    y_flat = y.reshape(NREL * NPAD, H)

    ei = jnp.stack(eis)
    rel_off = (jnp.arange(NREL, dtype=jnp.int32) * NPAD)[:, None, None]
    src = ei[:, 0, :].reshape(NREL, NW, EPW)
    dst = ei[:, 1, :].reshape(NREL, NW, EPW)
    pad = ((0, 0), (0, 0), (0, EPWP - EPW))
    src_all = (jnp.pad(src, pad) + rel_off).reshape(NREL, NW, NBLK, BE)
    dst_pad = N + jnp.arange(EPWP - EPW, dtype=jnp.int32)
    dst_p = jnp.concatenate(
        [dst, jnp.broadcast_to(dst_pad, (NREL, NW, EPWP - EPW))], axis=2)
    dst_all = dst_p.reshape(NREL, NW, NBLK, BE)

    agg, deg = _sc_segment_sums(src_all, dst_all, y_flat)

    out = _finish_all(x_pad, jnp.stack(w_selfs), agg, deg)
    return out[:, :N, :]


# speedup vs baseline: 1.3504x; 1.0147x over previous
import functools

import jax
import jax.numpy as jnp
from jax import lax
from jax.experimental import pallas as pl
from jax.experimental.pallas import tpu as pltpu
from jax.experimental.pallas import tpu_sc as plsc

N = 10000
D = 128
H = 64
E = 320000
NPAD = 10240
NC = 2
NS = 16
NW = NC * NS
EPW = E // NW
BE = 80
EPWP = 10000
NBLK = EPWP // BE
RB = 1024
NRB = NPAD // RB

VIEW_RELS = [3, 2, 2, 2, 1, 1, 1]
NREL = 12
VIEW_OF_REL = [0, 0, 0, 1, 1, 2, 2, 3, 3, 4, 5, 6]
REL_START = [0, 3, 5, 7, 9, 10, 11]


def _mm_body(x_ref, w_ref, y_ref):
    y_ref[...] = jnp.dot(x_ref[0], w_ref[0],
                         preferred_element_type=jnp.float32)[None]


def _rel_matmul(x_pad, w_rel):
    return pl.pallas_call(
        _mm_body,
        grid=(NREL, NRB),
        in_specs=[
            pl.BlockSpec(
                (1, RB, D),
                lambda r, b: (jnp.where(r < 3, 0,
                                        jnp.where(r < 9, (r - 3) // 2 + 1,
                                                  r - 5)), b, 0)),
            pl.BlockSpec((1, D, H), lambda r, b: (r, 0, 0)),
        ],
        out_specs=pl.BlockSpec((1, RB, H), lambda r, b: (r, b, 0)),
        out_shape=jax.ShapeDtypeStruct((NREL, NPAD, H), jnp.float32),
    )(x_pad, w_rel)


def _sc_body(src_hbm, dst_hbm, y_hbm, z2d_hbm, z1d_hbm, ones_hbm,
             agg_out, deg_out,
             acc_sh, deg_sh, srcb, dstb, rows, ones, sem, ssem, rsem):
    c = lax.axis_index("c")
    s = lax.axis_index("s")
    wid = c * NS + s
    rows_per_tile = NPAD // NS
    my = pl.ds(s * rows_per_tile, rows_per_tile)

    pltpu.sync_copy(ones_hbm, ones)
    for bk in (0, 1):
        pltpu.sync_copy(z2d_hbm, acc_sh.at[bk, my])
        pltpu.sync_copy(z1d_hbm, deg_sh.at[bk, my])
    for (zv, zk) in ((1, 2), (2, 2), (3, 2), (4, 1), (4, 2),
                     (5, 1), (5, 2), (6, 1), (6, 2)):
        pltpu.sync_copy(acc_sh.at[0, my], agg_out.at[zv, zk, c, my])
        pltpu.sync_copy(deg_sh.at[0, my], deg_out.at[zv, zk, c, my])
    plsc.subcore_barrier()

    def _vk_of(r):
        v_r = jnp.where(r < 3, 0,
                        jnp.where(r < 9, (r - 3) // 2 + 1, r - 5))
        k_r = jnp.where(r < 3, r, jnp.where(r < 9, (r - 3) % 2, 0))
        return v_r, k_r

    def rel_body(r, carry):
        bank = r % 2

        @pl.when(r >= 2)
        def _():
            v_p, k_p = _vk_of(r - 2)
            pltpu.make_async_copy(acc_sh.at[bank, my],
                                  agg_out.at[v_p, k_p, c, my], rsem).wait()
            pltpu.make_async_copy(deg_sh.at[bank, my],
                                  deg_out.at[v_p, k_p, c, my], rsem).wait()
            pltpu.sync_copy(z2d_hbm, acc_sh.at[bank, my])
            pltpu.sync_copy(z1d_hbm, deg_sh.at[bank, my])

        pltpu.sync_copy(src_hbm.at[r, wid], srcb)
        pltpu.sync_copy(dst_hbm.at[r, wid], dstb)
        plsc.subcore_barrier()

        pltpu.async_copy(y_hbm.at[srcb.at[0]], rows.at[0], sem)
        pltpu.async_copy(y_hbm.at[srcb.at[1]], rows.at[1], sem)

        def blk_body(j, carry2):
            b = j % 3
            pltpu.make_async_copy(y_hbm.at[srcb.at[j]], rows.at[b], sem).wait()
            pltpu.async_copy(rows.at[b], acc_sh.at[bank].at[dstb.at[j]],
                             ssem, add=True)
            pltpu.async_copy(ones.at[j], deg_sh.at[bank].at[dstb.at[j]],
                             ssem, add=True)

            @pl.when(j + 2 < NBLK)
            def _():
                @pl.when(j >= 1)
                def _():
                    pltpu.make_async_copy(
                        rows.at[(j - 1) % 3],
                        acc_sh.at[bank].at[dstb.at[j - 1]], ssem).wait()
                    pltpu.make_async_copy(
                        ones.at[j - 1], deg_sh.at[bank].at[dstb.at[j - 1]],
                        ssem).wait()
                pltpu.async_copy(y_hbm.at[srcb.at[j + 2]],
                                 rows.at[(j + 2) % 3], sem)

            return carry2

        lax.fori_loop(0, NBLK, blk_body, 0)
        for t in (NBLK - 3, NBLK - 2, NBLK - 1):
            pltpu.make_async_copy(rows.at[t % 3],
                                  acc_sh.at[bank].at[dstb.at[t]], ssem).wait()
            pltpu.make_async_copy(ones.at[t], deg_sh.at[bank].at[dstb.at[t]],
                                  ssem).wait()
        plsc.subcore_barrier()

        v_r, k_r = _vk_of(r)
        pltpu.async_copy(acc_sh.at[bank, my], agg_out.at[v_r, k_r, c, my],
                         rsem)
        pltpu.async_copy(deg_sh.at[bank, my], deg_out.at[v_r, k_r, c, my],
                         rsem)
        return carry

    lax.fori_loop(0, NREL, rel_body, 0)

    for rr, (v_t, k_t) in ((NREL - 2, (5, 0)), (NREL - 1, (6, 0))):
        pltpu.make_async_copy(acc_sh.at[rr % 2, my],
                              agg_out.at[v_t, k_t, c, my], rsem).wait()
        pltpu.make_async_copy(deg_sh.at[rr % 2, my],
                              deg_out.at[v_t, k_t, c, my], rsem).wait()


def _sc_segment_sums(src_all, dst_all, y_flat):
    rows_per_tile = NPAD // NS
    mesh = plsc.VectorSubcoreMesh(core_axis_name="c", subcore_axis_name="s",
                                  num_cores=NC, num_subcores=NS)
    kern = pl.kernel(
        _sc_body,
        out_type=(
            jax.ShapeDtypeStruct((7, 3, NC, NPAD, H), jnp.float32),
            jax.ShapeDtypeStruct((7, 3, NC, NPAD), jnp.float32),
        ),
        mesh=mesh,
        scratch_types=[
            pltpu.VMEM_SHARED((2, NPAD, H), jnp.float32),
            pltpu.VMEM_SHARED((2, NPAD), jnp.float32),
            pltpu.VMEM((NBLK, BE), jnp.int32),
            pltpu.VMEM((NBLK, BE), jnp.int32),
            pltpu.VMEM((3, BE, H), jnp.float32),
            pltpu.VMEM((NBLK, BE), jnp.float32),
            pltpu.SemaphoreType.DMA,
            pltpu.SemaphoreType.DMA,
            pltpu.SemaphoreType.DMA,
        ],
        compiler_params=pltpu.CompilerParams(use_tc_tiling_on_sc=False),
    )
    z2d = jnp.zeros((rows_per_tile, H), jnp.float32)
    z1d = jnp.zeros((rows_per_tile,), jnp.float32)
    ones = jnp.ones((NBLK, BE), jnp.float32)
    return kern(src_all, dst_all, y_flat, z2d, z1d, ones)


def _fin_body(x_ref, ws_ref, agg_ref, deg_ref, out_ref):
    h = jnp.dot(x_ref[0], ws_ref[0], preferred_element_type=jnp.float32)
    for k in range(3):
        a = agg_ref[0, k, 0] + agg_ref[0, k, 1]
        d = deg_ref[0, k, 0] + deg_ref[0, k, 1]
        h = h + a / jnp.maximum(d, 1.0)[:, None]
    out_ref[0] = h * (1.0 / (1.0 + jnp.exp(-h)))


def _finish_all(x_pad, w_self_all, agg, deg):
    return pl.pallas_call(
        _fin_body,
        grid=(7, NRB),
        in_specs=[
            pl.BlockSpec((1, RB, D), lambda v, b: (v, b, 0)),
            pl.BlockSpec((1, D, H), lambda v, b: (v, 0, 0)),
            pl.BlockSpec((1, 3, NC, RB, H), lambda v, b: (v, 0, 0, b, 0)),
            pl.BlockSpec((1, 3, NC, RB), lambda v, b: (v, 0, 0, b)),
        ],
        out_specs=pl.BlockSpec((1, RB, H), lambda v, b: (v, b, 0)),
        out_shape=jax.ShapeDtypeStruct((7, NPAD, H), jnp.float32),
    )(x_pad, w_self_all, agg, deg)


def kernel(x_coarse, ei_coarse_gene, ei_coarse_methy, ei_coarse_mirna,
           W_coarse_self, W_coarse_gene, W_coarse_methy, W_coarse_mirna,
           x_medium1, ei_medium1_gene, ei_medium1_methy,
           W_medium1_self, W_medium1_gene, W_medium1_methy,
           x_medium2, ei_medium2_gene, ei_medium2_mirna,
           W_medium2_self, W_medium2_gene, W_medium2_mirna,
           x_medium3, ei_medium3_methy, ei_medium3_mirna,
           W_medium3_self, W_medium3_methy, W_medium3_mirna,
           x_fine1, ei_fine1_gene, W_fine1_self, W_fine1_gene,
           x_fine2, ei_fine2_methy, W_fine2_self, W_fine2_methy,
           x_fine3, ei_fine3_mirna, W_fine3_self, W_fine3_mirna):
    xs = [x_coarse, x_medium1, x_medium2, x_medium3, x_fine1, x_fine2, x_fine3]
    eis = [ei_coarse_gene, ei_coarse_methy, ei_coarse_mirna,
           ei_medium1_gene, ei_medium1_methy,
           ei_medium2_gene, ei_medium2_mirna,
           ei_medium3_methy, ei_medium3_mirna,
           ei_fine1_gene, ei_fine2_methy, ei_fine3_mirna]
    w_selfs = [W_coarse_self, W_medium1_self, W_medium2_self, W_medium3_self,
               W_fine1_self, W_fine2_self, W_fine3_self]
    w_rels = [W_coarse_gene, W_coarse_methy, W_coarse_mirna,
              W_medium1_gene, W_medium1_methy,
              W_medium2_gene, W_medium2_mirna,
              W_medium3_methy, W_medium3_mirna,
              W_fine1_gene, W_fine2_methy, W_fine3_mirna]

    x_pad = jnp.pad(jnp.stack(xs), ((0, 0), (0, NPAD - N), (0, 0)))
    w_rel = jnp.stack(w_rels)

    y = _rel_matmul(x_pad, w_rel)
    y_flat = y.reshape(NREL * NPAD, H)

    ei = jnp.stack(eis)
    rel_off = (jnp.arange(NREL, dtype=jnp.int32) * NPAD)[:, None, None]
    src = ei[:, 0, :].reshape(NREL, NW, EPW)
    dst = ei[:, 1, :].reshape(NREL, NW, EPW)
    pad = ((0, 0), (0, 0), (0, EPWP - EPW))
    src_all = (jnp.pad(src, pad) + rel_off).reshape(NREL, NW, NBLK, BE)
    dst_pad = N + jnp.arange(EPWP - EPW, dtype=jnp.int32)
    dst_p = jnp.concatenate(
        [dst, jnp.broadcast_to(dst_pad, (NREL, NW, EPWP - EPW))], axis=2)
    dst_all = dst_p.reshape(NREL, NW, NBLK, BE)

    agg, deg = _sc_segment_sums(src_all, dst_all, y_flat)

    out = _finish_all(x_pad, jnp.stack(w_selfs), agg, deg)
    return out[:, :N, :]
